# baseline (speedup 1.0000x reference)
import functools
import math

import jax
import jax.numpy as jnp
from jax import lax
from jax.experimental import pallas as pl
from jax.experimental.pallas import tpu as pltpu
from jax.experimental.pallas import tpu_sc as plsc

D_EMBED = 512
HIDDEN = D_EMBED * 4
BATCH = 16384
TABLE_ROWS = 1000


_H_BLOCK = 256


def _mlp_body(x_ref, w1_ref, b1_ref, w2_ref, b2_ref, o_ref):
    j = pl.program_id(0)
    h = jnp.dot(x_ref[...], w1_ref[...], preferred_element_type=jnp.float32)
    h = h + b1_ref[...]
    h = h * jax.nn.sigmoid(h)
    part = jnp.dot(h, w2_ref[...], preferred_element_type=jnp.float32)

    @pl.when(j == 0)
    def _():
        o_ref[...] = part + b2_ref[...]

    @pl.when(j > 0)
    def _():
        o_ref[...] += part


def _mlp_table(table_p, W1, b1, W2, b2):
    grid = (HIDDEN // _H_BLOCK,)
    return pl.pallas_call(
        _mlp_body,
        grid=grid,
        in_specs=[
            pl.BlockSpec((TABLE_ROWS, D_EMBED), lambda j: (0, 0)),
            pl.BlockSpec((D_EMBED, _H_BLOCK), lambda j: (0, j)),
            pl.BlockSpec((1, _H_BLOCK), lambda j: (0, j)),
            pl.BlockSpec((_H_BLOCK, D_EMBED), lambda j: (j, 0)),
            pl.BlockSpec((1, D_EMBED), lambda j: (0, 0)),
        ],
        out_specs=pl.BlockSpec((TABLE_ROWS, D_EMBED), lambda j: (0, 0)),
        out_shape=jax.ShapeDtypeStruct((TABLE_ROWS, D_EMBED), jnp.float32),
    )(table_p, W1, b1, W2, b2)


_INFO = plsc.get_sparse_core_info()
_NC, _NS = _INFO.num_cores, _INFO.num_subcores
_NW = _NC * _NS
_B_PER_W = BATCH // _NW
_CHUNK = 64
_N_CHUNKS = _B_PER_W // _CHUNK

_SC_MESH = plsc.VectorSubcoreMesh(core_axis_name="c", subcore_axis_name="s")


_NBUF = 3


@functools.partial(
    pl.kernel,
    mesh=_SC_MESH,
    out_type=jax.ShapeDtypeStruct((BATCH, D_EMBED), jnp.float32),
    scratch_types=[
        pltpu.VMEM((_B_PER_W,), jnp.int32),
        pltpu.VMEM((_CHUNK, D_EMBED), jnp.float32),
        pltpu.VMEM((_CHUNK, D_EMBED), jnp.float32),
        pltpu.VMEM((_CHUNK, D_EMBED), jnp.float32),
        pltpu.SemaphoreType.DMA,
        pltpu.SemaphoreType.DMA,
        pltpu.SemaphoreType.DMA,
        pltpu.SemaphoreType.DMA,
        pltpu.SemaphoreType.DMA,
        pltpu.SemaphoreType.DMA,
    ],
)
def _gather(table_hbm, idx_hbm, out_hbm, idx_v, b0, b1, b2, g0, g1, g2, w0, w1, w2):
    wid = lax.axis_index("s") * _NC + lax.axis_index("c")
    base = wid * _B_PER_W
    pltpu.sync_copy(idx_hbm.at[pl.ds(base, _B_PER_W)], idx_v)
    bufs = (b0, b1, b2)
    gsems = (g0, g1, g2)
    wsems = (w0, w1, w2)

    def gather(c):
        return pltpu.async_copy(
            table_hbm.at[idx_v.at[pl.ds(c * _CHUNK, _CHUNK)]],
            bufs[c % _NBUF],
            gsems[c % _NBUF],
        )

    gcp = {0: gather(0), 1: gather(1)}
    wcp = {}
    for c in range(_N_CHUNKS):
        n = c + 2
        if n < _N_CHUNKS:
            if c >= 1:
                wcp[c - 1].wait()
            gcp[n] = gather(n)
        gcp[c].wait()
        wcp[c] = pltpu.async_copy(
            bufs[c % _NBUF],
            out_hbm.at[pl.ds(base + c * _CHUNK, _CHUNK)],
            wsems[c % _NBUF],
        )
    for c in range(_N_CHUNKS - 3, _N_CHUNKS):
        wcp[c].wait()


def kernel(t, table, W1, b1, W2, b2):
    mlp_tab = _mlp_table(
        table, W1, b1.reshape(1, HIDDEN), W2, b2.reshape(1, D_EMBED)
    )
    return _gather(mlp_tab, t)


# speedup vs baseline: 1.0780x; 1.0780x over previous
import functools
import math

import jax
import jax.numpy as jnp
from jax import lax
from jax.experimental import pallas as pl
from jax.experimental.pallas import tpu as pltpu
from jax.experimental.pallas import tpu_sc as plsc

D_EMBED = 512
HIDDEN = D_EMBED * 4
BATCH = 16384
TABLE_ROWS = 1000


_ROW_BLOCK = 200


def _mlp_body(x_ref, w1_ref, b1_ref, w2_ref, b2_ref, o_ref):
    x = x_ref[...]
    h = jnp.dot(x, w1_ref[...], preferred_element_type=jnp.float32)
    h = h + b1_ref[...]
    h = h * jax.nn.sigmoid(h)
    o = jnp.dot(h, w2_ref[...], preferred_element_type=jnp.float32)
    o_ref[...] = o + b2_ref[...]


def _mlp_table(table_p, W1, b1, W2, b2):
    grid = (TABLE_ROWS // _ROW_BLOCK,)
    return pl.pallas_call(
        _mlp_body,
        grid=grid,
        in_specs=[
            pl.BlockSpec((_ROW_BLOCK, D_EMBED), lambda i: (i, 0)),
            pl.BlockSpec((D_EMBED, HIDDEN), lambda i: (0, 0)),
            pl.BlockSpec((1, HIDDEN), lambda i: (0, 0)),
            pl.BlockSpec((HIDDEN, D_EMBED), lambda i: (0, 0)),
            pl.BlockSpec((1, D_EMBED), lambda i: (0, 0)),
        ],
        out_specs=pl.BlockSpec((_ROW_BLOCK, D_EMBED), lambda i: (i, 0)),
        out_shape=jax.ShapeDtypeStruct((TABLE_ROWS, D_EMBED), jnp.float32),
    )(table_p, W1, b1, W2, b2)


_INFO = plsc.get_sparse_core_info()
_NC, _NS = _INFO.num_cores, _INFO.num_subcores
_NW = _NC * _NS
_B_PER_W = BATCH // _NW
_CHUNK = 80
_CHUNKS = tuple((i * _CHUNK, _CHUNK) for i in range(6)) + ((480, 32),)

_SC_MESH = plsc.VectorSubcoreMesh(core_axis_name="c", subcore_axis_name="s")


_NBUF = 3


@functools.partial(
    pl.kernel,
    mesh=_SC_MESH,
    out_type=jax.ShapeDtypeStruct((BATCH, D_EMBED), jnp.float32),
    scratch_types=[
        pltpu.VMEM((_B_PER_W,), jnp.int32),
        pltpu.VMEM((_CHUNK, D_EMBED), jnp.float32),
        pltpu.VMEM((_CHUNK, D_EMBED), jnp.float32),
        pltpu.VMEM((_CHUNK, D_EMBED), jnp.float32),
        pltpu.SemaphoreType.DMA,
        pltpu.SemaphoreType.DMA,
        pltpu.SemaphoreType.DMA,
        pltpu.SemaphoreType.DMA,
        pltpu.SemaphoreType.DMA,
        pltpu.SemaphoreType.DMA,
    ],
)
def _gather(table_hbm, idx_hbm, out_hbm, idx_v, b0, b1, b2, g0, g1, g2, w0, w1, w2):
    wid = lax.axis_index("s") * _NC + lax.axis_index("c")
    base = wid * _B_PER_W
    pltpu.sync_copy(idx_hbm.at[pl.ds(base, _B_PER_W)], idx_v)
    bufs = (b0, b1, b2)
    gsems = (g0, g1, g2)
    wsems = (w0, w1, w2)

    def gather(c):
        off, sz = _CHUNKS[c]
        return pltpu.async_copy(
            table_hbm.at[idx_v.at[pl.ds(off, sz)]],
            bufs[c % _NBUF].at[pl.ds(0, sz)],
            gsems[c % _NBUF],
        )

    def write(c):
        off, sz = _CHUNKS[c]
        return pltpu.async_copy(
            bufs[c % _NBUF].at[pl.ds(0, sz)],
            out_hbm.at[pl.ds(base + off, sz)],
            wsems[c % _NBUF],
        )

    n_ch = len(_CHUNKS)
    gcp = {0: gather(0), 1: gather(1)}
    wcp = {}
    for c in range(n_ch):
        n = c + 2
        if n < n_ch:
            if c >= 1:
                wcp[c - 1].wait()
            gcp[n] = gather(n)
        gcp[c].wait()
        wcp[c] = write(c)
    for c in range(n_ch - 3, n_ch):
        wcp[c].wait()


def kernel(t, table, W1, b1, W2, b2):
    mlp_tab = _mlp_table(
        table, W1, b1.reshape(1, HIDDEN), W2, b2.reshape(1, D_EMBED)
    )
    return _gather(mlp_tab, t)
